# baseline (speedup 1.0000x reference)
import functools

import jax
import jax.numpy as jnp
from jax import lax
from jax.experimental import pallas as pl
from jax.experimental.pallas import tpu as pltpu
from jax.experimental.pallas import tpu_sc as plsc

N = 10000
E = 320000
D = 128
H = 128
B = 64
OUT = 16
DOC = 128

NC = 2
NS = 16
NW = NC * NS
EPW = E // NW
K = 80
NCHUNK = EPW // K
NPAD = 10240
RPT = NPAD // NS

BN = 10000
NB = N // BN

def _deg_body(dst_hbm, zeros_hbm, out_hbm, idx_v, hist_v):
    c = lax.axis_index("c")
    s = lax.axis_index("s")
    wid = c * NS + s
    pltpu.sync_copy(zeros_hbm, hist_v)
    pltpu.sync_copy(dst_hbm.at[wid], idx_v)
    ones = jnp.ones((16,), jnp.float32)

    def body(i, carry):
        for u in range(5):
            idx16 = idx_v[pl.ds(i * 80 + u * 16, 16)]
            plsc.addupdate_scatter(hist_v, [idx16], ones)
        return carry

    lax.fori_loop(0, EPW // 80, body, 0)
    pltpu.sync_copy(hist_v, out_hbm.at[wid])


def _agg_body(src_hbm, dst_hbm, g_hbm, out_hbm,
              src_v, dst_v, rows_a, rows_b, sem_a, sem_b, sem_sa, sem_sb,
              acc_sh):
    c = lax.axis_index("c")
    s = lax.axis_index("s")
    wid = c * NS + s
    zero16 = jnp.zeros((16,), jnp.float32)

    def zbody(t, carry):
        rows_a[t // 8, pl.ds((t % 8) * 16, 16)] = zero16
        return carry

    lax.fori_loop(0, K * 8, zbody, 0)

    def zcopy(q, carry):
        pltpu.sync_copy(rows_a, acc_sh.at[pl.ds(s * RPT + q * K, K)])
        return carry

    lax.fori_loop(0, RPT // K, zcopy, 0)
    pltpu.sync_copy(src_hbm.at[wid], src_v)
    pltpu.sync_copy(dst_hbm.at[wid], dst_v)
    plsc.subcore_barrier()

    def src_at(j):
        return src_v.at[pl.ds(pl.multiple_of(j * K, K), K)]

    pltpu.async_copy(g_hbm.at[src_at(0)], rows_a, sem_a)

    pltpu.async_copy(g_hbm.at[src_at(1)], rows_b, sem_b)

    def body(jj, carry):
        j0 = jj * 2
        j1 = j0 + 1
        pltpu.make_async_copy(g_hbm.at[src_at(j0)], rows_a, sem_a).wait()
        pltpu.async_copy(rows_a, acc_sh.at[dst_v.at[j0]], sem_sa, add=True)
        pltpu.make_async_copy(g_hbm.at[src_at(j1)], rows_b, sem_b).wait()
        pltpu.async_copy(rows_b, acc_sh.at[dst_v.at[j1]], sem_sb, add=True)
        pltpu.make_async_copy(rows_a, acc_sh.at[dst_v.at[j0]], sem_sa).wait()
        pltpu.async_copy(g_hbm.at[src_at(j0 + 2)], rows_a, sem_a)
        pltpu.make_async_copy(rows_b, acc_sh.at[dst_v.at[j1]], sem_sb).wait()

        @pl.when(j1 + 2 < NCHUNK)
        def _nextb():
            pltpu.async_copy(g_hbm.at[src_at(j1 + 2)], rows_b, sem_b)

        return carry

    lax.fori_loop(0, NCHUNK // 2, body, 0)
    pltpu.make_async_copy(g_hbm.at[src_at(NCHUNK - 1)], rows_a, sem_a).wait()
    pltpu.sync_copy(rows_a, acc_sh.at[dst_v.at[NCHUNK - 1]], add=True)
    plsc.subcore_barrier()
    pltpu.sync_copy(acc_sh.at[pl.ds(s * RPT, RPT)],
                    out_hbm.at[c, pl.ds(s * RPT, RPT)])


@functools.lru_cache(maxsize=None)
def _sc_kernels():
    mesh = plsc.VectorSubcoreMesh(
        core_axis_name="c", subcore_axis_name="s",
        num_cores=NC, num_subcores=NS)
    deg_kernel = pl.kernel(
        _deg_body,
        out_type=jax.ShapeDtypeStruct((NW, NPAD), jnp.float32),
        mesh=mesh,
        scratch_types=[
            pltpu.VMEM((EPW,), jnp.int32),
            pltpu.VMEM((NPAD,), jnp.float32),
        ],
        compiler_params=pltpu.CompilerParams(needs_layout_passes=False),
    )
    agg_kernel = pl.kernel(
        _agg_body,
        out_type=jax.ShapeDtypeStruct((NC, NPAD, H), jnp.float32),
        mesh=mesh,
        scratch_types=[
            pltpu.VMEM((EPW,), jnp.int32),
            pltpu.VMEM((NCHUNK, K), jnp.int32),
            pltpu.VMEM((K, H), jnp.float32),
            pltpu.VMEM((K, H), jnp.float32),
            pltpu.SemaphoreType.DMA,
            pltpu.SemaphoreType.DMA,
            pltpu.SemaphoreType.DMA,
            pltpu.SemaphoreType.DMA,
            pltpu.VMEM_SHARED((NPAD, H), jnp.float32),
        ],
    )
    return deg_kernel, agg_kernel


def _scale_body(x_ref, w_ref, degs_ref, g_ref, dinv_ref):
    d = jnp.sum(degs_ref[...], axis=1).reshape(BN, 1) + 1.0
    dv = lax.rsqrt(d)
    h = jnp.dot(x_ref[...], w_ref[...], preferred_element_type=jnp.float32)
    g_ref[...] = dv * h
    dinv_ref[...] = dv


def _phase_b(x, w_gcn, degs):
    return pl.pallas_call(
        _scale_body,
        grid=(NB,),
        in_specs=[
            pl.BlockSpec((BN, D), lambda i: (i, 0)),
            pl.BlockSpec((D, H), lambda i: (0, 0)),
            pl.BlockSpec((BN, NW), lambda i: (i, 0)),
        ],
        out_specs=[
            pl.BlockSpec((BN, H), lambda i: (i, 0)),
            pl.BlockSpec((BN, 1), lambda i: (i, 0)),
        ],
        out_shape=[
            jax.ShapeDtypeStruct((N, H), jnp.float32),
            jax.ShapeDtypeStruct((N, 1), jnp.float32),
        ],
    )(x, w_gcn, degs)


def _tail_body(accs_ref, g_ref, dinv_ref, batch_ref, bgcn_ref,
               doc_ref, wdoc_ref, bdoc_ref, lng_ref, lnb_ref,
               wfus_ref, bfus_ref, wtask_ref, btask_ref, wtime_ref, btime_ref,
               task_ref, time_ref, pooled_acc, count_acc):
    i = pl.program_id(0)

    @pl.when(i == 0)
    def _init():
        pooled_acc[...] = jnp.zeros_like(pooled_acc)
        count_acc[...] = jnp.zeros_like(count_acc)

    agg = accs_ref[0] + accs_ref[1] + g_ref[...]
    nodes = jax.nn.relu(dinv_ref[...] * agg + bgcn_ref[...])
    seg = lax.broadcasted_iota(jnp.int32, (BN, B), 1)
    mask = (batch_ref[...] == seg).astype(jnp.float32)
    pooled_acc[...] += lax.dot_general(
        mask, nodes, (((0,), (0,)), ((), ())),
        preferred_element_type=jnp.float32)
    count_acc[...] += lax.dot_general(
        mask, jnp.ones((BN, 1), jnp.float32), (((0,), (0,)), ((), ())),
        preferred_element_type=jnp.float32)

    @pl.when(i == NB - 1)
    def _final():
        pooled = pooled_acc[...] / jnp.maximum(count_acc[...], 1.0)
        doc_emb = jax.nn.relu(
            jnp.dot(doc_ref[...], wdoc_ref[...],
                    preferred_element_type=jnp.float32) + bdoc_ref[...])
        z = jnp.concatenate([pooled, doc_emb], axis=1)
        mu = jnp.mean(z, axis=-1, keepdims=True)
        var = jnp.mean((z - mu) ** 2, axis=-1, keepdims=True)
        z = (z - mu) * lax.rsqrt(var + 1e-5) * lng_ref[...] + lnb_ref[...]
        f = jax.nn.relu(
            jnp.dot(z, wfus_ref[...],
                    preferred_element_type=jnp.float32) + bfus_ref[...])
        task_ref[...] = jnp.dot(
            f, wtask_ref[...], preferred_element_type=jnp.float32) + btask_ref[...]
        time_ref[...] = jnp.dot(
            f, wtime_ref[...], preferred_element_type=jnp.float32) + btime_ref[...]


def _phase_d(accs, g, dinv, batch2, b_gcn, doc, w_doc, b_doc,
             ln_g, ln_b, w_fus, b_fus, w_task, b_task, w_time, b_time):
    full = lambda shape: pl.BlockSpec(shape, lambda i: tuple(0 for _ in shape))
    return pl.pallas_call(
        _tail_body,
        grid=(NB,),
        in_specs=[
            pl.BlockSpec((NC, BN, H), lambda i: (0, i, 0)),
            pl.BlockSpec((BN, H), lambda i: (i, 0)),
            pl.BlockSpec((BN, 1), lambda i: (i, 0)),
            pl.BlockSpec((BN, 1), lambda i: (i, 0)),
            full((1, H)),
            full((B, DOC)),
            full((DOC, H)),
            full((1, H)),
            full((1, 2 * H)),
            full((1, 2 * H)),
            full((2 * H, H)),
            full((1, H)),
            full((H, OUT)),
            full((1, OUT)),
            full((H, 1)),
            full((1, 1)),
        ],
        out_specs=[
            pl.BlockSpec((B, OUT), lambda i: (0, 0)),
            pl.BlockSpec((B, 1), lambda i: (0, 0)),
        ],
        out_shape=[
            jax.ShapeDtypeStruct((B, OUT), jnp.float32),
            jax.ShapeDtypeStruct((B, 1), jnp.float32),
        ],
        scratch_shapes=[
            pltpu.VMEM((B, H), jnp.float32),
            pltpu.VMEM((B, 1), jnp.float32),
        ],
    )(accs, g, dinv, batch2, b_gcn, doc, w_doc, b_doc,
      ln_g, ln_b, w_fus, b_fus, w_task, b_task, w_time, b_time)


def kernel(x, edge_index, batch, doc_features, W_gcn, b_gcn, W_doc, b_doc,
           ln_g, ln_b, W_fus, b_fus, W_task, b_task, W_time, b_time):
    src = edge_index[0].reshape(NW, EPW)
    dst = edge_index[1].reshape(NW, NCHUNK, K)
    dst_flat = edge_index[1].reshape(NW, EPW)
    zeros_npad = jnp.zeros((NPAD,), jnp.float32)

    deg_kernel, agg_kernel = _sc_kernels()
    degs = deg_kernel(dst_flat, zeros_npad)
    g, dinv = _phase_b(x, W_gcn, degs.T)
    accs = agg_kernel(src, dst, g)

    task, time = _phase_d(
        accs, g, dinv, batch.reshape(N, 1),
        b_gcn.reshape(1, H), doc_features, W_doc, b_doc.reshape(1, H),
        ln_g.reshape(1, 2 * H), ln_b.reshape(1, 2 * H),
        W_fus, b_fus.reshape(1, H), W_task, b_task.reshape(1, OUT),
        W_time, b_time.reshape(1, 1))
    return task, time


# speedup vs baseline: 1.0186x; 1.0186x over previous
import functools

import jax
import jax.numpy as jnp
from jax import lax
from jax.experimental import pallas as pl
from jax.experimental.pallas import tpu as pltpu
from jax.experimental.pallas import tpu_sc as plsc

N = 10000
E = 320000
D = 128
H = 128
B = 64
OUT = 16
DOC = 128

NC = 2
NS = 16
NW = NC * NS
EPW = E // NW
K = 80
NCHUNK = EPW // K
NPAD = 10240
RPT = NPAD // NS

BN = 5000
NB = N // BN

def _deg_body(dst_hbm, zeros_hbm, out_hbm, idx_v, hist_v):
    c = lax.axis_index("c")
    s = lax.axis_index("s")
    wid = c * NS + s
    pltpu.sync_copy(zeros_hbm, hist_v)
    pltpu.sync_copy(dst_hbm.at[wid], idx_v)
    ones = jnp.ones((16,), jnp.float32)

    def body(i, carry):
        for u in range(5):
            idx16 = idx_v[pl.ds(i * 80 + u * 16, 16)]
            plsc.addupdate_scatter(hist_v, [idx16], ones)
        return carry

    lax.fori_loop(0, EPW // 80, body, 0)
    pltpu.sync_copy(hist_v, out_hbm.at[wid])


def _agg_body(src_hbm, dst_hbm, g_hbm, out_hbm,
              src_v, dst_v, rows_a, rows_b, sem_a, sem_b, sem_sa, sem_sb,
              acc_sh):
    c = lax.axis_index("c")
    s = lax.axis_index("s")
    wid = c * NS + s
    zero16 = jnp.zeros((16,), jnp.float32)

    def zbody(t, carry):
        rows_a[t // 8, pl.ds((t % 8) * 16, 16)] = zero16
        return carry

    lax.fori_loop(0, K * 8, zbody, 0)

    def zcopy(q, carry):
        pltpu.sync_copy(rows_a, acc_sh.at[pl.ds(s * RPT + q * K, K)])
        return carry

    lax.fori_loop(0, RPT // K, zcopy, 0)
    pltpu.sync_copy(src_hbm.at[wid], src_v)
    pltpu.sync_copy(dst_hbm.at[wid], dst_v)
    plsc.subcore_barrier()

    def src_at(j):
        return src_v.at[pl.ds(pl.multiple_of(j * K, K), K)]

    pltpu.async_copy(g_hbm.at[src_at(0)], rows_a, sem_a)

    pltpu.async_copy(g_hbm.at[src_at(1)], rows_b, sem_b)

    def body(jj, carry):
        j0 = jj * 2
        j1 = j0 + 1
        pltpu.make_async_copy(g_hbm.at[src_at(j0)], rows_a, sem_a).wait()
        pltpu.async_copy(rows_a, acc_sh.at[dst_v.at[j0]], sem_sa, add=True)
        pltpu.make_async_copy(g_hbm.at[src_at(j1)], rows_b, sem_b).wait()
        pltpu.async_copy(rows_b, acc_sh.at[dst_v.at[j1]], sem_sb, add=True)
        pltpu.make_async_copy(rows_a, acc_sh.at[dst_v.at[j0]], sem_sa).wait()
        pltpu.async_copy(g_hbm.at[src_at(j0 + 2)], rows_a, sem_a)
        pltpu.make_async_copy(rows_b, acc_sh.at[dst_v.at[j1]], sem_sb).wait()

        @pl.when(j1 + 2 < NCHUNK)
        def _nextb():
            pltpu.async_copy(g_hbm.at[src_at(j1 + 2)], rows_b, sem_b)

        return carry

    lax.fori_loop(0, NCHUNK // 2, body, 0)
    pltpu.make_async_copy(g_hbm.at[src_at(NCHUNK - 1)], rows_a, sem_a).wait()
    pltpu.sync_copy(rows_a, acc_sh.at[dst_v.at[NCHUNK - 1]], add=True)
    plsc.subcore_barrier()
    pltpu.sync_copy(acc_sh.at[pl.ds(s * RPT, RPT)],
                    out_hbm.at[c, pl.ds(s * RPT, RPT)])


@functools.lru_cache(maxsize=None)
def _sc_kernels():
    mesh = plsc.VectorSubcoreMesh(
        core_axis_name="c", subcore_axis_name="s",
        num_cores=NC, num_subcores=NS)
    deg_kernel = pl.kernel(
        _deg_body,
        out_type=jax.ShapeDtypeStruct((NW, NPAD), jnp.float32),
        mesh=mesh,
        scratch_types=[
            pltpu.VMEM((EPW,), jnp.int32),
            pltpu.VMEM((NPAD,), jnp.float32),
        ],
        compiler_params=pltpu.CompilerParams(needs_layout_passes=False),
    )
    agg_kernel = pl.kernel(
        _agg_body,
        out_type=jax.ShapeDtypeStruct((NC, NPAD, H), jnp.float32),
        mesh=mesh,
        scratch_types=[
            pltpu.VMEM((EPW,), jnp.int32),
            pltpu.VMEM((NCHUNK, K), jnp.int32),
            pltpu.VMEM((K, H), jnp.float32),
            pltpu.VMEM((K, H), jnp.float32),
            pltpu.SemaphoreType.DMA,
            pltpu.SemaphoreType.DMA,
            pltpu.SemaphoreType.DMA,
            pltpu.SemaphoreType.DMA,
            pltpu.VMEM_SHARED((NPAD, H), jnp.float32),
        ],
    )
    return deg_kernel, agg_kernel


def _scale_body(x_ref, w_ref, degs_ref, g_ref, dinv_ref):
    d = jnp.sum(degs_ref[...], axis=1).reshape(BN, 1) + 1.0
    dv = lax.rsqrt(d)
    h = jnp.dot(x_ref[...], w_ref[...], preferred_element_type=jnp.float32)
    g_ref[...] = dv * h
    dinv_ref[...] = dv


def _phase_b(x, w_gcn, degs):
    return pl.pallas_call(
        _scale_body,
        grid=(NB,),
        in_specs=[
            pl.BlockSpec((BN, D), lambda i: (i, 0)),
            pl.BlockSpec((D, H), lambda i: (0, 0)),
            pl.BlockSpec((BN, NW), lambda i: (i, 0)),
        ],
        out_specs=[
            pl.BlockSpec((BN, H), lambda i: (i, 0)),
            pl.BlockSpec((BN, 1), lambda i: (i, 0)),
        ],
        out_shape=[
            jax.ShapeDtypeStruct((N, H), jnp.float32),
            jax.ShapeDtypeStruct((N, 1), jnp.float32),
        ],
    )(x, w_gcn, degs)


def _tail_body(accs_ref, g_ref, dinv_ref, batch_ref, bgcn_ref,
               doc_ref, wdoc_ref, bdoc_ref, lng_ref, lnb_ref,
               wfus_ref, bfus_ref, wtask_ref, btask_ref, wtime_ref, btime_ref,
               task_ref, time_ref, pooled_acc, count_acc):
    i = pl.program_id(0)

    @pl.when(i == 0)
    def _init():
        pooled_acc[...] = jnp.zeros_like(pooled_acc)
        count_acc[...] = jnp.zeros_like(count_acc)

    agg = accs_ref[0] + accs_ref[1] + g_ref[...]
    nodes = jax.nn.relu(dinv_ref[...] * agg + bgcn_ref[...])
    seg = lax.broadcasted_iota(jnp.int32, (BN, B), 1)
    mask = (batch_ref[...] == seg).astype(jnp.float32)
    pooled_acc[...] += lax.dot_general(
        mask, nodes, (((0,), (0,)), ((), ())),
        preferred_element_type=jnp.float32)
    count_acc[...] += lax.dot_general(
        mask, jnp.ones((BN, 1), jnp.float32), (((0,), (0,)), ((), ())),
        preferred_element_type=jnp.float32)

    @pl.when(i == NB - 1)
    def _final():
        pooled = pooled_acc[...] / jnp.maximum(count_acc[...], 1.0)
        doc_emb = jax.nn.relu(
            jnp.dot(doc_ref[...], wdoc_ref[...],
                    preferred_element_type=jnp.float32) + bdoc_ref[...])
        z = jnp.concatenate([pooled, doc_emb], axis=1)
        mu = jnp.mean(z, axis=-1, keepdims=True)
        var = jnp.mean((z - mu) ** 2, axis=-1, keepdims=True)
        z = (z - mu) * lax.rsqrt(var + 1e-5) * lng_ref[...] + lnb_ref[...]
        f = jax.nn.relu(
            jnp.dot(z, wfus_ref[...],
                    preferred_element_type=jnp.float32) + bfus_ref[...])
        task_ref[...] = jnp.dot(
            f, wtask_ref[...], preferred_element_type=jnp.float32) + btask_ref[...]
        time_ref[...] = jnp.dot(
            f, wtime_ref[...], preferred_element_type=jnp.float32) + btime_ref[...]


def _phase_d(accs, g, dinv, batch2, b_gcn, doc, w_doc, b_doc,
             ln_g, ln_b, w_fus, b_fus, w_task, b_task, w_time, b_time):
    full = lambda shape: pl.BlockSpec(shape, lambda i: tuple(0 for _ in shape))
    return pl.pallas_call(
        _tail_body,
        grid=(NB,),
        in_specs=[
            pl.BlockSpec((NC, BN, H), lambda i: (0, i, 0)),
            pl.BlockSpec((BN, H), lambda i: (i, 0)),
            pl.BlockSpec((BN, 1), lambda i: (i, 0)),
            pl.BlockSpec((BN, 1), lambda i: (i, 0)),
            full((1, H)),
            full((B, DOC)),
            full((DOC, H)),
            full((1, H)),
            full((1, 2 * H)),
            full((1, 2 * H)),
            full((2 * H, H)),
            full((1, H)),
            full((H, OUT)),
            full((1, OUT)),
            full((H, 1)),
            full((1, 1)),
        ],
        out_specs=[
            pl.BlockSpec((B, OUT), lambda i: (0, 0)),
            pl.BlockSpec((B, 1), lambda i: (0, 0)),
        ],
        out_shape=[
            jax.ShapeDtypeStruct((B, OUT), jnp.float32),
            jax.ShapeDtypeStruct((B, 1), jnp.float32),
        ],
        scratch_shapes=[
            pltpu.VMEM((B, H), jnp.float32),
            pltpu.VMEM((B, 1), jnp.float32),
        ],
    )(accs, g, dinv, batch2, b_gcn, doc, w_doc, b_doc,
      ln_g, ln_b, w_fus, b_fus, w_task, b_task, w_time, b_time)


def kernel(x, edge_index, batch, doc_features, W_gcn, b_gcn, W_doc, b_doc,
           ln_g, ln_b, W_fus, b_fus, W_task, b_task, W_time, b_time):
    src = edge_index[0].reshape(NW, EPW)
    dst = edge_index[1].reshape(NW, NCHUNK, K)
    dst_flat = edge_index[1].reshape(NW, EPW)
    zeros_npad = jnp.zeros((NPAD,), jnp.float32)

    deg_kernel, agg_kernel = _sc_kernels()
    degs = deg_kernel(dst_flat, zeros_npad)
    g, dinv = _phase_b(x, W_gcn, degs.T)
    accs = agg_kernel(src, dst, g)

    task, time = _phase_d(
        accs, g, dinv, batch.reshape(N, 1),
        b_gcn.reshape(1, H), doc_features, W_doc, b_doc.reshape(1, H),
        ln_g.reshape(1, 2 * H), ln_b.reshape(1, 2 * H),
        W_fus, b_fus.reshape(1, H), W_task, b_task.reshape(1, OUT),
        W_time, b_time.reshape(1, 1))
    return task, time
